# baseline (speedup 1.0000x reference)
import functools
import math

import jax
import jax.numpy as jnp
from jax import lax
from jax.experimental import pallas as pl
from jax.experimental.pallas import tpu as pltpu
from jax.experimental.pallas import tpu_sc as plsc

D = 128
T = 50
NSEQ = 4096
SCALE = math.sqrt(float(D))

_info = plsc.get_sparse_core_info()
_NC = _info.num_cores
_NS = _info.num_subcores
_NW = _NC * _NS
_L = _info.num_lanes

G = 4
CH = G * T
NBUF = 2
NSPLIT = 2

_mesh = plsc.VectorSubcoreMesh(core_axis_name="c", subcore_axis_name="s")


def _make_gather_scale(nseq):
    seq_per_w = nseq // _NW
    nch = seq_per_w // G

    @functools.partial(
        pl.kernel,
        mesh=_mesh,
        out_type=jax.ShapeDtypeStruct((nseq, T, D), jnp.float32),
        scratch_types=[
            pltpu.VMEM((seq_per_w * T,), jnp.int32),
            pltpu.VMEM((CH, D), jnp.float32),
            pltpu.VMEM((CH, D), jnp.float32),
            pltpu.SemaphoreType.DMA,
            pltpu.SemaphoreType.DMA,
        ],
        compiler_params=pltpu.CompilerParams(use_tc_tiling_on_sc=True),
    )
    def _gather_scale(idx_hbm, table_hbm, out_hbm, idx_v, buf0, buf1, gsem, osem):
        bufs = (buf0, buf1)
        wid = lax.axis_index("s") * _NC + lax.axis_index("c")
        base = wid * (seq_per_w * T)
        seq_base = wid * seq_per_w
        pltpu.sync_copy(idx_hbm.at[pl.ds(base, seq_per_w * T)], idx_v)

        def gather(c, buf):
            return pltpu.make_async_copy(
                table_hbm.at[idx_v.at[pl.ds(c * CH, CH)]], buf, gsem
            )

        def out_copies(c, buf):
            return [
                pltpu.make_async_copy(
                    buf.at[pl.ds(g * T, T)],
                    out_hbm.at[seq_base + c * G + g],
                    osem,
                )
                for g in range(G)
            ]

        def out_start(c, buf):
            for cp in out_copies(c, buf):
                cp.start()

        def out_wait(c, buf):
            for cp in out_copies(c, buf):
                cp.wait()

        def scale(buf):
            def row_body(r, c2):
                for j in range(D // _L):
                    sl = pl.ds(j * _L, _L)
                    buf[r, sl] = buf[r, sl] * SCALE
                return c2

            lax.fori_loop(0, CH, row_body, 0, unroll=2)

        gather(0, buf0).start()

        def chunk_body(p, carry):
            for b in range(NBUF):
                c = p * NBUF + b
                buf = bufs[b]
                bufn = bufs[(b + 1) % NBUF]
                gather(c, buf).wait()

                @pl.when(c + 1 < nch)
                def _():
                    @pl.when(c + 1 >= NBUF)
                    def _():
                        out_wait(c + 1 - NBUF, bufn)

                    gather(c + 1, bufn).start()

                scale(buf)
                out_start(c, buf)
            return carry

        lax.fori_loop(0, nch // NBUF, chunk_body, 0)
        out_wait(nch - NBUF, bufs[(nch - NBUF) % NBUF])
        out_wait(nch - 1, bufs[(nch - 1) % NBUF])

    return _gather_scale


_PART = NSEQ // NSPLIT
_gather_scale_part = _make_gather_scale(_PART)


def kernel(tokens, table):
    idx = tokens.reshape(-1).astype(jnp.int32)
    out = jnp.zeros((NSEQ, T, D), jnp.float32)
    for i in range(NSPLIT):
        part = _gather_scale_part(
            lax.dynamic_slice_in_dim(idx, i * _PART * T, _PART * T), table
        )
        out = lax.dynamic_update_slice_in_dim(out, part, i * _PART, axis=0)
    return out


# speedup vs baseline: 1.1278x; 1.1278x over previous
import functools
import math

import jax
import jax.numpy as jnp
from jax import lax
from jax.experimental import pallas as pl
from jax.experimental.pallas import tpu as pltpu
from jax.experimental.pallas import tpu_sc as plsc

D = 128
T = 50
NSEQ = 4096
SCALE = math.sqrt(float(D))

_info = plsc.get_sparse_core_info()
_NC = _info.num_cores
_NS = _info.num_subcores
_NW = _NC * _NS
_L = _info.num_lanes

G = 4
CH = G * T
NBUF = 2
NSPLIT = 2

_mesh = plsc.VectorSubcoreMesh(core_axis_name="c", subcore_axis_name="s")


def _make_gather_scale(nseq):
    seq_per_w = nseq // _NW
    nch = seq_per_w // G

    @functools.partial(
        pl.kernel,
        mesh=_mesh,
        out_type=jax.ShapeDtypeStruct((nseq, T, D), jnp.float32),
        scratch_types=[
            pltpu.VMEM((seq_per_w * T,), jnp.int32),
            pltpu.VMEM((CH, D), jnp.float32),
            pltpu.VMEM((CH, D), jnp.float32),
            pltpu.SemaphoreType.DMA,
            pltpu.SemaphoreType.DMA,
        ],
        compiler_params=pltpu.CompilerParams(use_tc_tiling_on_sc=True),
    )
    def _gather_scale(idx_hbm, table_hbm, out_hbm, idx_v, buf0, buf1, gsem, osem):
        bufs = (buf0, buf1)
        wid = lax.axis_index("s") * _NC + lax.axis_index("c")
        base = wid * (seq_per_w * T)
        seq_base = wid * seq_per_w
        pltpu.sync_copy(idx_hbm.at[pl.ds(base, seq_per_w * T)], idx_v)

        def gather(c, buf):
            return pltpu.make_async_copy(
                table_hbm.at[idx_v.at[pl.ds(c * CH, CH)]], buf, gsem
            )

        def out_copies(c, buf):
            return [
                pltpu.make_async_copy(
                    buf.at[pl.ds(g * T, T)],
                    out_hbm.at[seq_base + c * G + g],
                    osem,
                )
                for g in range(G)
            ]

        def out_start(c, buf):
            for cp in out_copies(c, buf):
                cp.start()

        def out_wait(c, buf):
            for cp in out_copies(c, buf):
                cp.wait()

        gather(0, buf0).start()

        def chunk_body(p, carry):
            for b in range(NBUF):
                c = p * NBUF + b
                buf = bufs[b]
                bufn = bufs[(b + 1) % NBUF]
                gather(c, buf).wait()

                @pl.when(c + 1 < nch)
                def _():
                    @pl.when(c + 1 >= NBUF)
                    def _():
                        out_wait(c + 1 - NBUF, bufn)

                    gather(c + 1, bufn).start()

                out_start(c, buf)
            return carry

        lax.fori_loop(0, nch // NBUF, chunk_body, 0)
        out_wait(nch - NBUF, bufs[(nch - NBUF) % NBUF])
        out_wait(nch - 1, bufs[(nch - 1) % NBUF])

    return _gather_scale


_gather_full = _make_gather_scale(NSEQ)

_SB = 128


def _scale_body(x_ref, o_ref):
    o_ref[...] = x_ref[...] * SCALE


_scale_tc = pl.pallas_call(
    _scale_body,
    out_shape=jax.ShapeDtypeStruct((NSEQ, T, D), jnp.float32),
    grid=(NSEQ // _SB,),
    in_specs=[pl.BlockSpec((_SB, T, D), lambda i: (i, 0, 0))],
    out_specs=pl.BlockSpec((_SB, T, D), lambda i: (i, 0, 0)),
)


def kernel(tokens, table):
    idx = tokens.reshape(-1).astype(jnp.int32)
    gathered = _gather_full(idx, table)
    return _scale_tc(gathered)


# speedup vs baseline: 2.8491x; 2.5261x over previous
import functools
import math

import jax
import jax.numpy as jnp
from jax import lax
from jax.experimental import pallas as pl
from jax.experimental.pallas import tpu as pltpu
from jax.experimental.pallas import tpu_sc as plsc

D = 128
T = 50
NSEQ = 4096
NROW = NSEQ * T
SCALE = math.sqrt(float(D))

_info = plsc.get_sparse_core_info()
_NC = _info.num_cores
_NS = _info.num_subcores
_NW = _NC * _NS
_L = _info.num_lanes

ROWS_PER_W = NROW // _NW
CH = 200
NCH = ROWS_PER_W // CH
NBUF = 2

_mesh = plsc.VectorSubcoreMesh(core_axis_name="c", subcore_axis_name="s")


@functools.partial(
    pl.kernel,
    mesh=_mesh,
    out_type=jax.ShapeDtypeStruct((NROW, D), jnp.float32),
    scratch_types=[
        pltpu.VMEM((ROWS_PER_W,), jnp.int32),
        pltpu.VMEM((CH, D), jnp.float32),
        pltpu.VMEM((CH, D), jnp.float32),
        pltpu.SemaphoreType.DMA,
        pltpu.SemaphoreType.DMA,
    ],
    compiler_params=pltpu.CompilerParams(use_tc_tiling_on_sc=True),
)
def _gather_scale(idx_hbm, table_hbm, out_hbm, idx_v, buf0, buf1, gsem, osem):
    bufs = (buf0, buf1)
    wid = lax.axis_index("s") * _NC + lax.axis_index("c")
    base = wid * ROWS_PER_W
    pltpu.sync_copy(idx_hbm.at[pl.ds(base, ROWS_PER_W)], idx_v)

    def gather(c, buf):
        return pltpu.make_async_copy(
            table_hbm.at[idx_v.at[pl.ds(c * CH, CH)]], buf, gsem
        )

    def out_copy(c, buf):
        return pltpu.make_async_copy(
            buf, out_hbm.at[pl.ds(base + c * CH, CH)], osem
        )

    def scale(buf):
        def row_body(r, c2):
            for j in range(D // _L):
                sl = pl.ds(j * _L, _L)
                buf[r, sl] = buf[r, sl] * SCALE
            return c2

        lax.fori_loop(0, CH, row_body, 0, unroll=2)

    gather(0, buf0).start()

    def chunk_body(p, carry):
        for b in range(NBUF):
            c = p * NBUF + b
            buf = bufs[b]
            bufn = bufs[(b + 1) % NBUF]
            gather(c, buf).wait()

            @pl.when(c + 1 < NCH)
            def _():
                @pl.when(c + 1 >= NBUF)
                def _():
                    out_copy(c + 1 - NBUF, bufn).wait()

                gather(c + 1, bufn).start()

            scale(buf)
            out_copy(c, buf).start()
        return carry

    lax.fori_loop(0, NCH // NBUF, chunk_body, 0)
    out_copy(NCH - NBUF, bufs[(NCH - NBUF) % NBUF]).wait()
    out_copy(NCH - 1, bufs[(NCH - 1) % NBUF]).wait()


def kernel(tokens, table):
    idx = tokens.T.reshape(-1).astype(jnp.int32)
    flat = _gather_scale(idx, table)
    return flat.reshape(T, NSEQ, D).transpose(1, 0, 2)


# speedup vs baseline: 2.9105x; 1.0215x over previous
import functools
import math

import jax
import jax.numpy as jnp
from jax import lax
from jax.experimental import pallas as pl
from jax.experimental.pallas import tpu as pltpu
from jax.experimental.pallas import tpu_sc as plsc

D = 128
T = 50
NSEQ = 4096
NROW = NSEQ * T
SCALE = math.sqrt(float(D))

_info = plsc.get_sparse_core_info()
_NC = _info.num_cores
_NS = _info.num_subcores
_NW = _NC * _NS
_L = _info.num_lanes

ROWS_PER_W = NROW // _NW
CH = 400
NCH = ROWS_PER_W // CH
NBUF = 2

_mesh = plsc.VectorSubcoreMesh(core_axis_name="c", subcore_axis_name="s")


@functools.partial(
    pl.kernel,
    mesh=_mesh,
    out_type=jax.ShapeDtypeStruct((NROW, D), jnp.float32),
    scratch_types=[
        pltpu.VMEM((ROWS_PER_W,), jnp.int32),
        pltpu.VMEM((CH, D), jnp.float32),
        pltpu.VMEM((CH, D), jnp.float32),
        pltpu.SemaphoreType.DMA,
        pltpu.SemaphoreType.DMA,
    ],
    compiler_params=pltpu.CompilerParams(use_tc_tiling_on_sc=True),
)
def _gather_scale(idx_hbm, table_hbm, out_hbm, idx_v, buf0, buf1, gsem, osem):
    bufs = (buf0, buf1)
    wid = lax.axis_index("s") * _NC + lax.axis_index("c")
    base = wid * ROWS_PER_W
    pltpu.sync_copy(idx_hbm.at[pl.ds(base, ROWS_PER_W)], idx_v)

    def gather(c, buf):
        return pltpu.make_async_copy(
            table_hbm.at[idx_v.at[pl.ds(c * CH, CH)]], buf, gsem
        )

    def out_copy(c, buf):
        return pltpu.make_async_copy(
            buf, out_hbm.at[pl.ds(base + c * CH, CH)], osem
        )

    def scale(buf):
        def row_body(r, c2):
            for j in range(D // _L):
                sl = pl.ds(j * _L, _L)
                buf[r, sl] = buf[r, sl] * SCALE
            return c2

        lax.fori_loop(0, CH, row_body, 0, unroll=2)

    gather(0, buf0).start()

    def chunk_body(p, carry):
        for b in range(NBUF):
            c = p * NBUF + b
            buf = bufs[b]
            bufn = bufs[(b + 1) % NBUF]
            gather(c, buf).wait()

            @pl.when(c + 1 < NCH)
            def _():
                @pl.when(c + 1 >= NBUF)
                def _():
                    out_copy(c + 1 - NBUF, bufn).wait()

                gather(c + 1, bufn).start()

            scale(buf)
            out_copy(c, buf).start()
        return carry

    lax.fori_loop(0, NCH // NBUF, chunk_body, 0)
    out_copy(NCH - NBUF, bufs[(NCH - NBUF) % NBUF]).wait()
    out_copy(NCH - 1, bufs[(NCH - 1) % NBUF]).wait()


def kernel(tokens, table):
    idx = tokens.T.reshape(-1).astype(jnp.int32)
    flat = _gather_scale(idx, table)
    return flat.reshape(T, NSEQ, D).transpose(1, 0, 2)


# speedup vs baseline: 2.9922x; 1.0281x over previous
import functools
import math

import jax
import jax.numpy as jnp
from jax import lax
from jax.experimental import pallas as pl
from jax.experimental.pallas import tpu as pltpu
from jax.experimental.pallas import tpu_sc as plsc

D = 128
T = 50
NSEQ = 4096
NROW = NSEQ * T
SCALE = math.sqrt(float(D))

_info = plsc.get_sparse_core_info()
_NC = _info.num_cores
_NS = _info.num_subcores
_NW = _NC * _NS
_L = _info.num_lanes

ROWS_PER_W = NROW // _NW
CH = 200
NCH = ROWS_PER_W // CH
NBUF = 4

_mesh = plsc.VectorSubcoreMesh(core_axis_name="c", subcore_axis_name="s")


@functools.partial(
    pl.kernel,
    mesh=_mesh,
    out_type=jax.ShapeDtypeStruct((NROW, D), jnp.float32),
    scratch_types=(
        [pltpu.VMEM((ROWS_PER_W,), jnp.int32)]
        + [pltpu.VMEM((CH, D), jnp.float32) for _ in range(NBUF)]
        + [pltpu.SemaphoreType.DMA, pltpu.SemaphoreType.DMA]
    ),
    compiler_params=pltpu.CompilerParams(use_tc_tiling_on_sc=True),
)
def _gather_scale(idx_hbm, table_hbm, out_hbm, idx_v, *rest):
    bufs = rest[:NBUF]
    gsem, osem = rest[NBUF], rest[NBUF + 1]
    wid = lax.axis_index("s") * _NC + lax.axis_index("c")
    base = wid * ROWS_PER_W
    pltpu.sync_copy(idx_hbm.at[pl.ds(base, ROWS_PER_W)], idx_v)

    def gather(c, buf):
        return pltpu.make_async_copy(
            table_hbm.at[idx_v.at[pl.ds(c * CH, CH)]], buf, gsem
        )

    def out_copy(c, buf):
        return pltpu.make_async_copy(
            buf, out_hbm.at[pl.ds(base + c * CH, CH)], osem
        )

    def scale(buf):
        def row_body(r, c2):
            for j in range(D // _L):
                sl = pl.ds(j * _L, _L)
                buf[r, sl] = buf[r, sl] * SCALE
            return c2

        lax.fori_loop(0, CH, row_body, 0, unroll=2)

    for k in range(NBUF - 1):
        gather(k, bufs[k]).start()

    def chunk_body(p, carry):
        for b in range(NBUF):
            c = p * NBUF + b
            buf = bufs[b]
            gather(c, buf).wait()

            t = c + NBUF - 1
            tb = bufs[(b + NBUF - 1) % NBUF]

            @pl.when(t < NCH)
            def _():
                @pl.when(t >= NBUF)
                def _():
                    out_copy(t - NBUF, tb).wait()

                gather(t, tb).start()

            scale(buf)
            out_copy(c, buf).start()
        return carry

    lax.fori_loop(0, NCH // NBUF, chunk_body, 0)
    for k in range(NBUF):
        c = NCH - NBUF + k
        out_copy(c, bufs[c % NBUF]).wait()


def kernel(tokens, table):
    idx = tokens.T.reshape(-1).astype(jnp.int32)
    flat = _gather_scale(idx, table)
    return flat.reshape(T, NSEQ, D).transpose(1, 0, 2)
